# baseline (speedup 1.0000x reference)
import functools

import jax
import jax.numpy as jnp
from jax import lax
from jax.experimental import pallas as pl
from jax.experimental.pallas import tpu as pltpu
from jax.experimental.pallas import tpu_sc as plsc

_BATCH = 16384
_D = 64
_NC = 2
_NS = 16
_NW = _NC * _NS
_BPW = _BATCH // _NW
_TP = 5120
_NB = 196
_TGRID = 49
_RROWS = _TGRID * _TP


def _to_bf16_bits(x):
    xu = lax.bitcast_convert_type(x, jnp.uint32)
    lsb = lax.shift_right_logical(xu, jnp.uint32(16)) & jnp.uint32(1)
    return lax.shift_right_logical(
        xu + jnp.uint32(0x7FFF) + lsb, jnp.uint32(16))


def _pack_body(u0_ref, u1_ref, u2_ref, u3_ref, i0_ref, i1_ref, i2_ref,
               i3_ref, ru_ref, ri_ref):
    def packT(hi_ref, lo_ref):
        w = lax.shift_left(_to_bf16_bits(hi_ref[...]), jnp.uint32(16))
        w = w | _to_bf16_bits(lo_ref[...])
        return jnp.swapaxes(lax.bitcast_convert_type(w, jnp.float32), 0, 1)

    ru_ref[:, :_D] = packT(u0_ref, u1_ref)
    ru_ref[:, _D:] = packT(u2_ref, u3_ref)
    ri_ref[:, :_D] = packT(i0_ref, i1_ref)
    ri_ref[:, _D:] = packT(i2_ref, i3_ref)


def _pack(ut, it):
    def src(k):
        return pl.BlockSpec((_D, _TP), lambda n, k=k: (0, n + k * _TGRID))

    out = pl.BlockSpec((_TP, 2 * _D), lambda n: (n, 0))
    return pl.pallas_call(
        _pack_body,
        grid=(_TGRID,),
        in_specs=[src(0), src(1), src(2), src(3)] * 2,
        out_specs=[out, out],
        out_shape=[
            jax.ShapeDtypeStruct((_RROWS, 2 * _D), jnp.float32),
            jax.ShapeDtypeStruct((_RROWS, 2 * _D), jnp.float32),
        ],
    )(ut, ut, ut, ut, it, it, it, it)


def _gather_body(uidx_hbm, iidx_hbm, ru_hbm, ri_hbm, u_out, i_out,
                 idx_v, rows_v, sem):
    wid = lax.axis_index("s") * _NC + lax.axis_index("c")
    base = wid * _BPW
    for idx_hbm, tab_hbm, out_hbm in ((uidx_hbm, ru_hbm, u_out),
                                      (iidx_hbm, ri_hbm, i_out)):
        pltpu.sync_copy(idx_hbm.at[pl.ds(base, _BPW)], idx_v)
        pltpu.async_copy(tab_hbm.at[idx_v], rows_v, sem).wait()
        pltpu.sync_copy(rows_v, out_hbm.at[pl.ds(base, _BPW)])


_gather = pl.kernel(
    _gather_body,
    out_type=(
        jax.ShapeDtypeStruct((_BATCH, 2 * _D), jnp.float32),
        jax.ShapeDtypeStruct((_BATCH, 2 * _D), jnp.float32),
    ),
    mesh=plsc.VectorSubcoreMesh(core_axis_name="c", subcore_axis_name="s"),
    scratch_types=[
        pltpu.VMEM((_BPW,), jnp.int32),
        pltpu.VMEM((_BPW, 2 * _D), jnp.float32),
        pltpu.SemaphoreType.DMA,
    ],
)


def _mlp_body(u2_ref, i2_ref, qu_ref, qi_ref, w1a_ref, w1b_ref, b1_ref,
              w2_ref, b2_ref, w3_ref, b3_ref, out_ref):
    def select(x_ref, q_ref):
        xu = lax.bitcast_convert_type(x_ref[...], jnp.uint32)
        w01 = xu[:, :_D]
        w23 = xu[:, _D:]
        q = q_ref[...]

        def unhi(w):
            return lax.bitcast_convert_type(
                w & jnp.uint32(0xFFFF0000), jnp.float32)

        def unlo(w):
            return lax.bitcast_convert_type(
                lax.shift_left(w, jnp.uint32(16)), jnp.float32)

        acc = unhi(w01) * (q == 0.0)
        acc = acc + unlo(w01) * (q == 1.0)
        acc = acc + unhi(w23) * (q == 2.0)
        acc = acc + unlo(w23) * (q == 3.0)
        return acc

    u = select(u2_ref, qu_ref)
    i = select(i2_ref, qi_ref)
    h = jnp.dot(u, w1a_ref[...], preferred_element_type=jnp.float32)
    h = h + jnp.dot(i, w1b_ref[...], preferred_element_type=jnp.float32)
    h = jnp.maximum(h + b1_ref[...], 0.0)
    h = jnp.maximum(
        jnp.dot(h, w2_ref[...], preferred_element_type=jnp.float32) + b2_ref[...], 0.0)
    out_ref[...] = (
        jnp.dot(h, w3_ref[...], preferred_element_type=jnp.float32) + b3_ref[...])


def _mlp(u2, i2, qu, qi, W1a, W1b, b1r, W2, b2r, W3, b3r, tile=4096):
    grid = (_BATCH // tile,)
    full = lambda shape: pl.BlockSpec(shape, lambda n: (0, 0))
    return pl.pallas_call(
        _mlp_body,
        grid=grid,
        in_specs=[
            pl.BlockSpec((tile, 2 * _D), lambda n: (n, 0)),
            pl.BlockSpec((tile, 2 * _D), lambda n: (n, 0)),
            pl.BlockSpec((tile, 1), lambda n: (n, 0)),
            pl.BlockSpec((tile, 1), lambda n: (n, 0)),
            full(W1a.shape), full(W1b.shape), full(b1r.shape),
            full(W2.shape), full(b2r.shape), full(W3.shape), full(b3r.shape),
        ],
        out_specs=pl.BlockSpec((tile, 1), lambda n: (n, 0)),
        out_shape=jax.ShapeDtypeStruct((_BATCH, 1), jnp.float32),
    )(u2, i2, qu, qi, W1a, W1b, b1r, W2, b2r, W3, b3r)


@jax.jit
def kernel(indexes, user_table, item_table, W1, b1, W2, b2, W3, b3):
    ut = jnp.swapaxes(user_table, 0, 1)
    it = jnp.swapaxes(item_table, 0, 1)
    ru, ri = _pack(ut, it)

    def remap(r):
        m = r // _TP
        q = r - m * _TP
        quarter = m // _TGRID
        p = (m - quarter * _TGRID) * _TP + q
        return p, quarter.astype(jnp.float32).reshape(_BATCH, 1)

    pu, qu = remap(indexes[0])
    pi, qi = remap(indexes[1])
    u2, i2 = _gather(pu, pi, ru, ri)
    return _mlp(u2, i2, qu, qi, W1[:_D], W1[_D:], b1.reshape(1, -1), W2,
                b2.reshape(1, -1), W3, b3.reshape(1, -1))


# speedup vs baseline: 1.0033x; 1.0033x over previous
import functools

import jax
import jax.numpy as jnp
from jax import lax
from jax.experimental import pallas as pl
from jax.experimental.pallas import tpu as pltpu
from jax.experimental.pallas import tpu_sc as plsc

_BATCH = 16384
_D = 64
_NC = 2
_NS = 16
_NW = _NC * _NS
_BPW = _BATCH // _NW
_TP = 5120
_NB = 196
_TGRID = 49
_RROWS = _TGRID * _TP


def _to_bf16_bits(x):
    xu = lax.bitcast_convert_type(x, jnp.uint32)
    lsb = lax.shift_right_logical(xu, jnp.uint32(16)) & jnp.uint32(1)
    return lax.shift_right_logical(
        xu + jnp.uint32(0x7FFF) + lsb, jnp.uint32(16))


def _pack_body(u0_ref, u1_ref, u2_ref, u3_ref, i0_ref, i1_ref, i2_ref,
               i3_ref, ru_ref, ri_ref):
    def packT(hi_ref, lo_ref):
        w = lax.shift_left(_to_bf16_bits(hi_ref[...]), jnp.uint32(16))
        w = w | _to_bf16_bits(lo_ref[...])
        return jnp.swapaxes(lax.bitcast_convert_type(w, jnp.float32), 0, 1)

    ru_ref[:, :_D] = packT(u0_ref, u1_ref)
    ru_ref[:, _D:] = packT(u2_ref, u3_ref)
    ri_ref[:, :_D] = packT(i0_ref, i1_ref)
    ri_ref[:, _D:] = packT(i2_ref, i3_ref)


def _pack(ut, it):
    def src(k):
        return pl.BlockSpec((_D, _TP), lambda n, k=k: (0, n + k * _TGRID))

    out = pl.BlockSpec((_TP, 2 * _D), lambda n: (n, 0))
    return pl.pallas_call(
        _pack_body,
        grid=(_TGRID,),
        in_specs=[src(0), src(1), src(2), src(3)] * 2,
        out_specs=[out, out],
        out_shape=[
            jax.ShapeDtypeStruct((_RROWS, 2 * _D), jnp.float32),
            jax.ShapeDtypeStruct((_RROWS, 2 * _D), jnp.float32),
        ],
    )(ut, ut, ut, ut, it, it, it, it)


def _gather_body(uidx_hbm, iidx_hbm, ru_hbm, ri_hbm, u_out, i_out,
                 idx_v, rows_v, sem):
    wid = lax.axis_index("s") * _NC + lax.axis_index("c")
    base = wid * _BPW
    for idx_hbm, tab_hbm, out_hbm in ((uidx_hbm, ru_hbm, u_out),
                                      (iidx_hbm, ri_hbm, i_out)):
        pltpu.sync_copy(idx_hbm.at[pl.ds(base, _BPW)], idx_v)
        pltpu.async_copy(tab_hbm.at[idx_v], rows_v, sem).wait()
        pltpu.sync_copy(rows_v, out_hbm.at[pl.ds(base, _BPW)])


_gather = pl.kernel(
    _gather_body,
    out_type=(
        jax.ShapeDtypeStruct((_BATCH, 2 * _D), jnp.float32),
        jax.ShapeDtypeStruct((_BATCH, 2 * _D), jnp.float32),
    ),
    mesh=plsc.VectorSubcoreMesh(core_axis_name="c", subcore_axis_name="s"),
    scratch_types=[
        pltpu.VMEM((_BPW,), jnp.int32),
        pltpu.VMEM((_BPW, 2 * _D), jnp.float32),
        pltpu.SemaphoreType.DMA,
    ],
)


def _mlp_body(u2_ref, i2_ref, qu_ref, qi_ref, w1a_ref, w1b_ref, b1_ref,
              w2_ref, b2_ref, w3_ref, b3_ref, out_ref):
    def select(x_ref, q_ref):
        xu = lax.bitcast_convert_type(x_ref[...], jnp.uint32)
        w01 = xu[:, :_D]
        w23 = xu[:, _D:]
        q = q_ref[...]

        def unhi(w):
            return lax.bitcast_convert_type(
                w & jnp.uint32(0xFFFF0000), jnp.float32)

        def unlo(w):
            return lax.bitcast_convert_type(
                lax.shift_left(w, jnp.uint32(16)), jnp.float32)

        acc = unhi(w01) * (q == 0.0)
        acc = acc + unlo(w01) * (q == 1.0)
        acc = acc + unhi(w23) * (q == 2.0)
        acc = acc + unlo(w23) * (q == 3.0)
        return acc

    u = select(u2_ref, qu_ref)
    i = select(i2_ref, qi_ref)
    h = jnp.dot(u, w1a_ref[...], preferred_element_type=jnp.float32)
    h = h + jnp.dot(i, w1b_ref[...], preferred_element_type=jnp.float32)
    h = jnp.maximum(h + b1_ref[...], 0.0)
    h = jnp.maximum(
        jnp.dot(h, w2_ref[...], preferred_element_type=jnp.float32) + b2_ref[...], 0.0)
    out_ref[...] = (
        jnp.dot(h, w3_ref[...], preferred_element_type=jnp.float32) + b3_ref[...])


def _mlp(u2, i2, qu, qi, W1a, W1b, b1r, W2, b2r, W3, b3r, tile=2048):
    grid = (_BATCH // tile,)
    full = lambda shape: pl.BlockSpec(shape, lambda n: (0, 0))
    return pl.pallas_call(
        _mlp_body,
        grid=grid,
        in_specs=[
            pl.BlockSpec((tile, 2 * _D), lambda n: (n, 0)),
            pl.BlockSpec((tile, 2 * _D), lambda n: (n, 0)),
            pl.BlockSpec((tile, 1), lambda n: (n, 0)),
            pl.BlockSpec((tile, 1), lambda n: (n, 0)),
            full(W1a.shape), full(W1b.shape), full(b1r.shape),
            full(W2.shape), full(b2r.shape), full(W3.shape), full(b3r.shape),
        ],
        out_specs=pl.BlockSpec((tile, 1), lambda n: (n, 0)),
        out_shape=jax.ShapeDtypeStruct((_BATCH, 1), jnp.float32),
    )(u2, i2, qu, qi, W1a, W1b, b1r, W2, b2r, W3, b3r)


@jax.jit
def kernel(indexes, user_table, item_table, W1, b1, W2, b2, W3, b3):
    ut = jnp.swapaxes(user_table, 0, 1)
    it = jnp.swapaxes(item_table, 0, 1)
    ru, ri = _pack(ut, it)

    def remap(r):
        m = r // _TP
        q = r - m * _TP
        quarter = m // _TGRID
        p = (m - quarter * _TGRID) * _TP + q
        return p, quarter.astype(jnp.float32).reshape(_BATCH, 1)

    pu, qu = remap(indexes[0])
    pi, qi = remap(indexes[1])
    u2, i2 = _gather(pu, pi, ru, ri)
    return _mlp(u2, i2, qu, qi, W1[:_D], W1[_D:], b1.reshape(1, -1), W2,
                b2.reshape(1, -1), W3, b3.reshape(1, -1))
